# Initial kernel scaffold; baseline (speedup 1.0000x reference)
import jax
import jax.numpy as jnp
from jax.experimental import pallas as pl


def kernel(input, target):
    raise NotImplementedError("write your pallas kernel here")


# speedup vs baseline: 25.4390x; 25.4390x over previous
import jax
import jax.numpy as jnp
from jax import lax
from jax.experimental import pallas as pl

_GAMMA = 0.5
_TOPK = 512
_SENT = -3.0e38
_IMIN = -(2 ** 31)
_KEY_LO = -2139095040
_KEY_HI = 2139095039


def _f_neg(v):
    s = 1.0 / (1.0 + jnp.exp(-v))
    return jnp.log(1.0 - jnp.minimum(s, jnp.float32(0.999)))


def _tc_body(x_ref, tgt_ref, pos_ref, neg_ref):
    i = pl.program_id(0)
    x = x_ref[...]
    tgt = tgt_ref[0]
    cols = lax.broadcasted_iota(jnp.int32, x.shape, 1)
    excl = cols == tgt
    pos_logit = jnp.sum(jnp.where(excl, x, 0.0), axis=1, keepdims=True)
    xm = jnp.where(excl, _SENT, x)

    kb = lax.bitcast_convert_type(xm, jnp.int32)
    key = jnp.where(kb >= 0, kb, _IMIN - kb)

    lo = jnp.full((x.shape[0], 1), _KEY_LO, jnp.int32)
    hi = jnp.full((x.shape[0], 1), _KEY_HI, jnp.int32)

    def body(_, c):
        lo, hi = c
        mid = (lo >> 1) + (hi >> 1) + (lo & hi & 1) + ((lo ^ hi) & 1)
        cnt = jnp.sum(jnp.where(key >= mid, 1, 0), axis=1, keepdims=True)
        pred = cnt >= _TOPK
        return (jnp.where(pred, mid, lo), jnp.where(pred, hi, mid - 1))

    lo, hi = lax.fori_loop(0, 32, body, (lo, hi))
    t_key = lo
    n_above = jnp.sum(jnp.where(key > t_key, 1, 0), axis=1, keepdims=True)
    t_b = jnp.where(t_key >= 0, t_key, _IMIN - t_key)
    t_f = lax.bitcast_convert_type(t_b, jnp.float32)

    f_all = _f_neg(xm)
    neg_blk = (jnp.sum(jnp.where(key > t_key, f_all, 0.0), keepdims=True)
               + jnp.sum((_TOPK - n_above).astype(jnp.float32) * _f_neg(t_f),
                         keepdims=True))

    p = jnp.maximum(1.0 / (1.0 + jnp.exp(-pos_logit)), jnp.float32(0.001))
    pos_blk = jnp.sum(jnp.log(p), keepdims=True)

    @pl.when(i == 0)
    def _init():
        pos_ref[...] = jnp.zeros((1, 1), jnp.float32)
        neg_ref[...] = jnp.zeros((1, 1), jnp.float32)

    pos_ref[...] += pos_blk
    neg_ref[...] += neg_blk


def kernel(input, target):
    b, c = input.shape
    rows = 8
    nblk = b // rows
    tgt3 = target.astype(jnp.int32).reshape(nblk, rows, 1)
    pos_sum, neg_sum = pl.pallas_call(
        _tc_body,
        grid=(nblk,),
        in_specs=[
            pl.BlockSpec((rows, c), lambda i: (i, 0)),
            pl.BlockSpec((1, rows, 1), lambda i: (i, 0, 0)),
        ],
        out_specs=[
            pl.BlockSpec((1, 1), lambda i: (0, 0)),
            pl.BlockSpec((1, 1), lambda i: (0, 0)),
        ],
        out_shape=[
            jax.ShapeDtypeStruct((1, 1), jnp.float32),
            jax.ShapeDtypeStruct((1, 1), jnp.float32),
        ],
    )(input, tgt3)
    pos_loss = -pos_sum[0, 0] / b
    neg_loss = -neg_sum[0, 0] / (b * _TOPK)
    return pos_loss * _GAMMA + neg_loss * (1.0 - _GAMMA)


# speedup vs baseline: 30.5473x; 1.2008x over previous
import functools

import jax
import jax.numpy as jnp
from jax import lax
from jax.experimental import pallas as pl
from jax.experimental.pallas import tpu as pltpu
from jax.experimental.pallas import tpu_sc as plsc

_GAMMA = 0.5
_TOPK = 512
_NCORE = 2
_NSUB = 16
_NW = _NCORE * _NSUB
_CAP = 2048
_CANDBUF = _CAP + 32
_SENT = -3.0e38
_IMIN = -(2 ** 31)
_KEY_LO = -2139095040
_KEY_HI = 2139095039
_SUB_NV = 512
_SUB_RANK = 96
_NUDGE = 1 << 20


def _ceil_avg(lo, hi):
    return (lo >> 1) + (hi >> 1) + (lo & hi & 1) + ((lo ^ hi) & 1)


def _key_to_fvec(mid):
    mv = jnp.full((16,), mid, jnp.int32)
    bv = jnp.where(mv >= 0, mv, _IMIN - mv)
    return plsc.bitcast(bv, jnp.float32)


def _sc_topk(x, tgt):
    b, c = x.shape
    nv = c // 16
    rpw = b // _NW
    mesh = plsc.VectorSubcoreMesh(core_axis_name="c", subcore_axis_name="s")

    @functools.partial(
        pl.kernel,
        out_type=[
            jax.ShapeDtypeStruct((b, _TOPK), jnp.float32),
            jax.ShapeDtypeStruct((b,), jnp.float32),
        ],
        mesh=mesh,
        compiler_params=pltpu.CompilerParams(needs_layout_passes=False),
        scratch_types=[
            pltpu.VMEM((c,), jnp.float32),
            pltpu.VMEM((_CANDBUF,), jnp.float32),
            pltpu.VMEM((_TOPK,), jnp.float32),
            pltpu.VMEM((rpw,), jnp.int32),
            pltpu.VMEM((rpw,), jnp.float32),
            pltpu.VMEM((16,), jnp.int32),
        ],
    )
    def sc_kernel(x_hbm, tgt_hbm, outneg_hbm, outpos_hbm,
                  row_v, cand_v, out_v, tgt_v, pos_v, cnt_v):
        wid = lax.axis_index("s") * _NCORE + lax.axis_index("c")
        base = wid * rpw
        iota = lax.iota(jnp.int32, 16)
        lane0 = iota == 0
        zero16 = jnp.zeros((16,), jnp.int32)
        sent_vec = jnp.full((16,), _SENT, jnp.float32)

        def count_ref(ref, nvec, tvec):
            def cbody(i, acc):
                xv = ref[pl.ds(i * 16, 16)]
                return acc + plsc.all_reduce_population_count(xv >= tvec)
            return jnp.max(lax.fori_loop(0, nvec, cbody, zero16))

        def sweep_compact(tvec):
            def sbody(i, off):
                xv = row_v[pl.ds(i * 16, 16)]
                m = xv >= tvec
                cs = plsc.cumsum(jnp.where(m, jnp.int32(1), jnp.int32(0)))
                idx = jnp.minimum(off + cs - 1, jnp.int32(_CAP + 15))
                plsc.store_scatter(cand_v, [idx], xv, mask=m)
                return off + plsc.all_reduce_population_count(m)
            return jnp.max(lax.fori_loop(0, nv, sbody, zero16))

        def strict_compact(tvec):
            def sbody(i, off):
                xv = row_v[pl.ds(i * 16, 16)]
                m = xv > tvec
                cs = plsc.cumsum(jnp.where(m, jnp.int32(1), jnp.int32(0)))
                idx = jnp.minimum(off + cs - 1, jnp.int32(_CAP + 15))
                plsc.store_scatter(cand_v, [idx], xv, mask=m)
                return off + plsc.all_reduce_population_count(m)
            return jnp.max(lax.fori_loop(0, nv, sbody, zero16))

        def bootstrap():
            def cond(st):
                lo, hi = st
                return lo < hi
            def bbody(st):
                lo, hi = st
                mid = _ceil_avg(lo, hi)
                cc = count_ref(row_v, _SUB_NV, _key_to_fvec(mid))
                return (jnp.where(cc >= _SUB_RANK, mid, lo),
                        jnp.where(cc >= _SUB_RANK, hi, mid - 1))
            lo, _ = lax.while_loop(
                cond, bbody, (jnp.int32(_KEY_LO), jnp.int32(_KEY_HI)))
            return lo

        def process_row(r, warm):
            def cond(st):
                lo, hi, mid, cc, hit = st
                return jnp.logical_and(jnp.logical_not(hit), lo < hi)

            def obody(st):
                lo, hi, mid, _, _ = st
                cc = sweep_compact(_key_to_fvec(mid))
                ge = cc >= _TOPK
                hit = jnp.logical_and(ge, cc <= _CAP)
                lo2 = jnp.where(ge, mid, lo)
                hi2 = jnp.where(ge, hi, mid - 1)
                mid2 = jnp.where(hit, mid, _ceil_avg(lo2, hi2))
                return (lo2, hi2, mid2, cc, hit)

            mid0 = jnp.clip(warm, jnp.int32(_KEY_LO + 1), jnp.int32(_KEY_HI))
            st0 = (jnp.int32(_KEY_LO), jnp.int32(_KEY_HI), mid0,
                   jnp.int32(0), jnp.bool_(False))
            lo, hi, mid, cc, hit = lax.while_loop(cond, obody, st0)

            cnt_v[...] = jnp.full((16,), cc, jnp.int32)

            @pl.when(jnp.logical_not(hit))
            def _():
                a = strict_compact(_key_to_fvec(lo))
                cnt_v[...] = jnp.full((16,), a, jnp.int32)

            c2 = jnp.max(cnt_v[...])
            lo2 = jnp.where(hit, mid, lo)
            hi2 = jnp.where(hit, hi, lo)

            plsc.store_scatter(cand_v, [jnp.full((16,), c2, jnp.int32) + iota],
                               sent_vec)
            nv2 = (c2 + 15) // 16

            def rcond(st):
                rlo, rhi = st
                return rlo < rhi
            def rbody(st):
                rlo, rhi = st
                rmid = _ceil_avg(rlo, rhi)
                rc = count_ref(cand_v, nv2, _key_to_fvec(rmid))
                return (jnp.where(rc >= _TOPK, rmid, rlo),
                        jnp.where(rc >= _TOPK, rhi, rmid - 1))
            t_key, _ = lax.while_loop(rcond, rbody, (lo2, hi2))

            tvec = _key_to_fvec(t_key)
            def fbody(v, _):
                out_v[pl.ds(v * 16, 16)] = tvec
                return 0
            lax.fori_loop(0, _TOPK // 16, fbody, 0)

            def ebody(i, off):
                xv = cand_v[pl.ds(i * 16, 16)]
                m = xv > tvec
                cs = plsc.cumsum(jnp.where(m, jnp.int32(1), jnp.int32(0)))
                idx = jnp.minimum(off + cs - 1, jnp.int32(_TOPK - 1))
                plsc.store_scatter(out_v, [idx], xv, mask=m)
                return off + plsc.all_reduce_population_count(m)
            lax.fori_loop(0, nv2, ebody, zero16)

            pltpu.sync_copy(out_v, outneg_hbm.at[r])
            return t_key

        def prep_row(j):
            jv = jnp.full((16,), j, jnp.int32)
            tg = plsc.load_gather(tgt_v, [jv])
            xpos = plsc.load_gather(row_v, [tg])
            plsc.store_scatter(pos_v, [jv], xpos, mask=lane0)
            plsc.store_scatter(row_v, [tg], sent_vec, mask=lane0)

        pltpu.sync_copy(tgt_hbm.at[pl.ds(base, rpw)], tgt_v)

        pltpu.sync_copy(x_hbm.at[base], row_v)
        prep_row(jnp.int32(0))
        t0 = process_row(base, bootstrap())

        def rowbody(j, warm):
            pltpu.sync_copy(x_hbm.at[base + j], row_v)
            prep_row(j)
            t = process_row(base + j, warm)
            return t - _NUDGE

        lax.fori_loop(1, rpw, rowbody, t0 - _NUDGE)
        pltpu.sync_copy(pos_v, outpos_hbm.at[pl.ds(base, rpw)])

    return sc_kernel(x, tgt)


def _f_neg(v):
    s = 1.0 / (1.0 + jnp.exp(-v))
    return jnp.log(1.0 - jnp.minimum(s, jnp.float32(0.999)))


def _tc_reduce_body(neg_ref, pos_ref, out_ref):
    neg_sum = jnp.sum(_f_neg(neg_ref[...]), keepdims=True)
    p = jnp.maximum(1.0 / (1.0 + jnp.exp(-pos_ref[...])), jnp.float32(0.001))
    pos_sum = jnp.sum(jnp.log(p), keepdims=True)
    b = pos_ref.shape[0] * pos_ref.shape[1]
    out_ref[...] = (_GAMMA * (-pos_sum / b)
                    + (1.0 - _GAMMA) * (-neg_sum / (b * _TOPK)))


def kernel(input, target):
    b, c = input.shape
    neg_tops, pos_logits = _sc_topk(input, target.astype(jnp.int32))
    loss = pl.pallas_call(
        _tc_reduce_body,
        out_shape=jax.ShapeDtypeStruct((1, 1), jnp.float32),
    )(neg_tops, pos_logits.reshape(8, b // 8))
    return loss[0, 0]


# speedup vs baseline: 46.5138x; 1.5227x over previous
import functools

import jax
import jax.numpy as jnp
from jax import lax
from jax.experimental import pallas as pl
from jax.experimental.pallas import tpu as pltpu
from jax.experimental.pallas import tpu_sc as plsc

_GAMMA = 0.5
_TOPK = 512
_NCORE = 2
_NSUB = 16
_NW = _NCORE * _NSUB
_CAP = 1024
_BCAP = 16 * _CAP
_SENT = -3.0e38
_IMIN = -(2 ** 31)
_KEY_LO = -2139095040
_KEY_HI = 2139095039
_SUB_NV = 512
_SUB_RANK = 64
_NUDGE = 1 << 19
_U = 25


def _ceil_avg(lo, hi):
    return (lo >> 1) + (hi >> 1) + (lo & hi & 1) + ((lo ^ hi) & 1)


def _key_to_fvec(mid):
    mv = jnp.full((16,), mid, jnp.int32)
    bv = jnp.where(mv >= 0, mv, _IMIN - mv)
    return plsc.bitcast(bv, jnp.float32)


def _sc_topk(x, tgt):
    b, c = x.shape
    nv = c // 16
    rpw = b // _NW
    mesh = plsc.VectorSubcoreMesh(core_axis_name="c", subcore_axis_name="s")

    @functools.partial(
        pl.kernel,
        out_type=[
            jax.ShapeDtypeStruct((b, _TOPK), jnp.float32),
            jax.ShapeDtypeStruct((b,), jnp.float32),
        ],
        mesh=mesh,
        compiler_params=pltpu.CompilerParams(needs_layout_passes=False),
        scratch_types=[
            pltpu.VMEM((c,), jnp.float32),
            pltpu.VMEM((_BCAP + 32,), jnp.float32),
            pltpu.VMEM((_TOPK,), jnp.float32),
            pltpu.VMEM((rpw,), jnp.int32),
            pltpu.VMEM((rpw,), jnp.float32),
            pltpu.VMEM((16,), jnp.int32),
            pltpu.VMEM((16,), jnp.int32),
        ],
    )
    def sc_kernel(x_hbm, tgt_hbm, outneg_hbm, outpos_hbm,
                  row_v, blk_v, out_v, tgt_v, pos_v, cnt_v, key_v):
        wid = lax.axis_index("s") * _NCORE + lax.axis_index("c")
        base = wid * rpw
        iota = lax.iota(jnp.int32, 16)
        lane0 = iota == 0
        zero16 = jnp.zeros((16,), jnp.int32)
        sent_vec = jnp.full((16,), _SENT, jnp.float32)

        def count_ref(ref, nvec, tvec):
            def cbody(i, acc):
                xv = ref[pl.ds(i * 16, 16)]
                return acc + plsc.all_reduce_population_count(xv >= tvec)
            return jnp.max(lax.fori_loop(0, nvec, cbody, zero16))

        def sweep_blocks(tvec):
            def sbody(it, carry):
                off, cnt = carry
                for u in range(_U):
                    xv = row_v[pl.ds((it * _U + u) * 16, 16)]
                    m = xv >= tvec
                    pc = plsc.all_reduce_population_count(m)
                    plsc.store_scatter(
                        blk_v, [jnp.minimum(off, jnp.int32(_BCAP - 16)) + iota],
                        xv)
                    off = off + jnp.where(pc > 0, jnp.int32(16), jnp.int32(0))
                    cnt = cnt + pc
                return off, cnt
            off, cnt = lax.fori_loop(0, nv // _U, sbody, (zero16, zero16))
            return jnp.max(cnt), jnp.max(off)

        def exact_compact(tvec, nwords):
            def p2(k, carry):
                off2, mx = carry
                xv = blk_v[pl.ds(k * 16, 16)]
                m = xv >= tvec
                cs = plsc.cumsum(jnp.where(m, jnp.int32(1), jnp.int32(0)))
                plsc.store_scatter(blk_v, [off2 + cs - 1], xv, mask=m)
                off2 = off2 + plsc.all_reduce_population_count(m)
                return off2, jnp.maximum(mx, jnp.where(m, xv, sent_vec))
            return lax.fori_loop(0, nwords // 16, p2, (zero16, sent_vec))

        def strict_compact(tvec):
            def sbody(i, off):
                xv = row_v[pl.ds(i * 16, 16)]
                m = xv > tvec
                cs = plsc.cumsum(jnp.where(m, jnp.int32(1), jnp.int32(0)))
                idx = jnp.minimum(off + cs - 1, jnp.int32(_BCAP + 15))
                plsc.store_scatter(blk_v, [idx], xv, mask=m)
                return off + plsc.all_reduce_population_count(m)
            return lax.fori_loop(0, nv, sbody, zero16)

        def bootstrap():
            def cond(st):
                lo, hi = st
                return lo < hi
            def bbody(st):
                lo, hi = st
                mid = _ceil_avg(lo, hi)
                cc = count_ref(row_v, _SUB_NV, _key_to_fvec(mid))
                return (jnp.where(cc >= _SUB_RANK, mid, lo),
                        jnp.where(cc >= _SUB_RANK, hi, mid - 1))
            lo, _ = lax.while_loop(
                cond, bbody, (jnp.int32(_KEY_LO), jnp.int32(_KEY_HI)))
            return lo

        def process_row(r, warm):
            def cond(st):
                lo, hi, mid, cc, nw, hit = st
                return jnp.logical_and(jnp.logical_not(hit), lo < hi)

            def obody(st):
                lo, hi, mid, _, _, _ = st
                cc, nw = sweep_blocks(_key_to_fvec(mid))
                ge = cc >= _TOPK
                hit = jnp.logical_and(ge, cc <= _CAP)
                lo2 = jnp.where(ge, mid, lo)
                hi2 = jnp.where(ge, hi, mid - 1)
                mid2 = jnp.where(hit, mid, _ceil_avg(lo2, hi2))
                return (lo2, hi2, mid2, cc, nw, hit)

            mid0 = jnp.clip(warm, jnp.int32(_KEY_LO + 1), jnp.int32(_KEY_HI))
            st0 = (jnp.int32(_KEY_LO), jnp.int32(_KEY_HI), mid0,
                   jnp.int32(0), jnp.int32(0), jnp.bool_(False))
            lo, hi, mid, cc, nw, hit = lax.while_loop(cond, obody, st0)

            cnt_v[...] = zero16
            key_v[...] = jnp.full((16,), lo, jnp.int32)

            @pl.when(hit)
            def _():
                c2v, mxv = exact_compact(_key_to_fvec(mid), nw)
                cnt_v[...] = c2v
                bk = plsc.bitcast(mxv, jnp.int32)
                kk = jnp.where(bk >= 0, bk, _IMIN - bk)
                key_v[...] = jnp.minimum(jnp.full((16,), hi, jnp.int32), kk)

            @pl.when(jnp.logical_not(hit))
            def _():
                cnt_v[...] = strict_compact(_key_to_fvec(lo))

            c2 = jnp.max(cnt_v[...])
            hi2 = jnp.max(key_v[...])
            lo2 = jnp.where(hit, mid, lo)

            plsc.store_scatter(blk_v, [jnp.full((16,), c2, jnp.int32) + iota],
                               sent_vec)
            nv2 = (c2 + 15) // 16

            def rcond(st):
                rlo, rhi = st
                return rlo < rhi
            def rbody(st):
                rlo, rhi = st
                rmid = _ceil_avg(rlo, rhi)
                rc = count_ref(blk_v, nv2, _key_to_fvec(rmid))
                return (jnp.where(rc >= _TOPK, rmid, rlo),
                        jnp.where(rc >= _TOPK, rhi, rmid - 1))
            t_key, _ = lax.while_loop(rcond, rbody, (lo2, hi2))

            tvec = _key_to_fvec(t_key)
            def fbody(v, _):
                out_v[pl.ds(v * 16, 16)] = tvec
                return 0
            lax.fori_loop(0, _TOPK // 16, fbody, 0)

            def ebody(i, off):
                xv = blk_v[pl.ds(i * 16, 16)]
                m = xv > tvec
                cs = plsc.cumsum(jnp.where(m, jnp.int32(1), jnp.int32(0)))
                idx = jnp.minimum(off + cs - 1, jnp.int32(_TOPK - 1))
                plsc.store_scatter(out_v, [idx], xv, mask=m)
                return off + plsc.all_reduce_population_count(m)
            lax.fori_loop(0, nv2, ebody, zero16)

            pltpu.sync_copy(out_v, outneg_hbm.at[r])
            return t_key

        def prep_row(j):
            jv = jnp.full((16,), j, jnp.int32)
            tg = plsc.load_gather(tgt_v, [jv])
            xpos = plsc.load_gather(row_v, [tg])
            plsc.store_scatter(pos_v, [jv], xpos, mask=lane0)
            plsc.store_scatter(row_v, [tg], sent_vec, mask=lane0)

        pltpu.sync_copy(tgt_hbm.at[pl.ds(base, rpw)], tgt_v)

        pltpu.sync_copy(x_hbm.at[base], row_v)
        prep_row(jnp.int32(0))
        t0 = process_row(base, bootstrap())

        def rowbody(j, warm):
            pltpu.sync_copy(x_hbm.at[base + j], row_v)
            prep_row(j)
            t = process_row(base + j, warm)
            return t - _NUDGE

        lax.fori_loop(1, rpw, rowbody, t0 - _NUDGE)
        pltpu.sync_copy(pos_v, outpos_hbm.at[pl.ds(base, rpw)])

    return sc_kernel(x, tgt)


def _f_neg(v):
    s = 1.0 / (1.0 + jnp.exp(-v))
    return jnp.log(1.0 - jnp.minimum(s, jnp.float32(0.999)))


def _tc_reduce_body(neg_ref, pos_ref, out_ref):
    neg_sum = jnp.sum(_f_neg(neg_ref[...]), keepdims=True)
    p = jnp.maximum(1.0 / (1.0 + jnp.exp(-pos_ref[...])), jnp.float32(0.001))
    pos_sum = jnp.sum(jnp.log(p), keepdims=True)
    b = pos_ref.shape[0] * pos_ref.shape[1]
    out_ref[...] = (_GAMMA * (-pos_sum / b)
                    + (1.0 - _GAMMA) * (-neg_sum / (b * _TOPK)))


def kernel(input, target):
    b, c = input.shape
    neg_tops, pos_logits = _sc_topk(input, target.astype(jnp.int32))
    loss = pl.pallas_call(
        _tc_reduce_body,
        out_shape=jax.ShapeDtypeStruct((1, 1), jnp.float32),
    )(neg_tops, pos_logits.reshape(8, b // 8))
    return loss[0, 0]


# speedup vs baseline: 81.5934x; 1.7542x over previous
import functools

import jax
import jax.numpy as jnp
from jax import lax
from jax.experimental import pallas as pl
from jax.experimental.pallas import tpu as pltpu
from jax.experimental.pallas import tpu_sc as plsc

_GAMMA = 0.5
_TOPK = 512
_NCORE = 2
_NSUB = 16
_NW = _NCORE * _NSUB
_CAP = 1024
_BCAP = 16 * _CAP
_SENT = -3.0e38
_IMIN = -(2 ** 31)
_KEY_LO = -2139095040
_KEY_HI = 2139095039
_SUB_NV = 256
_SUB_RANK = 48
_NUDGE = 1 << 19
_U = 10


def _ceil_avg(lo, hi):
    return (lo >> 1) + (hi >> 1) + (lo & hi & 1) + ((lo ^ hi) & 1)


def _key_to_fvec(mid):
    mv = jnp.full((16,), mid, jnp.int32)
    bv = jnp.where(mv >= 0, mv, _IMIN - mv)
    return plsc.bitcast(bv, jnp.float32)


def _sc_topk(x, tgt):
    b, c = x.shape
    nv = c // 16
    rpw = b // _NW
    mesh = plsc.VectorSubcoreMesh(core_axis_name="c", subcore_axis_name="s")

    @functools.partial(
        pl.kernel,
        out_type=[
            jax.ShapeDtypeStruct((b, _TOPK), jnp.float32),
            jax.ShapeDtypeStruct((b,), jnp.float32),
        ],
        mesh=mesh,
        compiler_params=pltpu.CompilerParams(needs_layout_passes=False),
        scratch_types=[
            pltpu.VMEM((c,), jnp.float32),
            pltpu.VMEM((_BCAP + 80,), jnp.float32),
            pltpu.VMEM((_TOPK,), jnp.float32),
            pltpu.VMEM((rpw,), jnp.int32),
            pltpu.VMEM((rpw,), jnp.float32),
            pltpu.VMEM((16,), jnp.int32),
            pltpu.VMEM((16,), jnp.int32),
        ],
    )
    def sc_kernel(x_hbm, tgt_hbm, outneg_hbm, outpos_hbm,
                  row_v, blk_v, out_v, tgt_v, pos_v, cnt_v, key_v):
        wid = lax.axis_index("s") * _NCORE + lax.axis_index("c")
        base = wid * rpw
        iota = lax.iota(jnp.int32, 16)
        lane0 = iota == 0
        zero16 = jnp.zeros((16,), jnp.int32)
        sent_vec = jnp.full((16,), _SENT, jnp.float32)

        def count_ref(ref, ngroups, tvec):
            def cbody(i, acc):
                ps = []
                for u in range(4):
                    xv = ref[pl.ds((i * 4 + u) * 16, 16)]
                    ps.append(plsc.all_reduce_population_count(xv >= tvec))
                return acc + ((ps[0] + ps[1]) + (ps[2] + ps[3]))
            return jnp.max(lax.fori_loop(0, ngroups, cbody, zero16))

        def sweep_blocks(tvec):
            def sbody(it, off):
                xs, advs = [], []
                for u in range(_U):
                    xv = row_v[pl.ds((it * _U + u) * 16, 16)]
                    m = xv >= tvec
                    pc = plsc.all_reduce_population_count(m)
                    xs.append(xv)
                    advs.append(jnp.where(pc > 0, jnp.int32(16), jnp.int32(0)))
                for u in range(_U):
                    idx = (off & jnp.int32(_BCAP - 1)) + iota
                    plsc.store_scatter(blk_v, [idx], xs[u])
                    off = off + advs[u]
                return off
            off = lax.fori_loop(0, nv // _U, sbody, zero16)
            return jnp.max(off)

        def exact_compact(tvec, nwords):
            def p2(k, carry):
                off2, mx = carry
                xv = blk_v[pl.ds(k * 16, 16)]
                m = xv >= tvec
                cs = plsc.cumsum(jnp.where(m, jnp.int32(1), jnp.int32(0)))
                plsc.store_scatter(blk_v, [off2 + cs - 1], xv, mask=m)
                off2 = off2 + plsc.all_reduce_population_count(m)
                return off2, jnp.maximum(mx, jnp.where(m, xv, sent_vec))
            return lax.fori_loop(0, nwords // 16, p2, (zero16, sent_vec))

        def strict_compact(tvec):
            def sbody(i, off):
                xv = row_v[pl.ds(i * 16, 16)]
                m = xv > tvec
                cs = plsc.cumsum(jnp.where(m, jnp.int32(1), jnp.int32(0)))
                idx = jnp.minimum(off + cs - 1, jnp.int32(_BCAP + 15))
                plsc.store_scatter(blk_v, [idx], xv, mask=m)
                return off + plsc.all_reduce_population_count(m)
            return lax.fori_loop(0, nv, sbody, zero16)

        def bootstrap():
            def cond(st):
                lo, hi = st
                return lo < hi
            def bbody(st):
                lo, hi = st
                mid = _ceil_avg(lo, hi)
                cc = count_ref(row_v, _SUB_NV // 4, _key_to_fvec(mid))
                return (jnp.where(cc >= _SUB_RANK, mid, lo),
                        jnp.where(cc >= _SUB_RANK, hi, mid - 1))
            lo, _ = lax.while_loop(
                cond, bbody, (jnp.int32(_KEY_LO), jnp.int32(_KEY_HI)))
            return lo

        def process_row(r, warm):
            def cond(st):
                lo, hi, mid, hit = st
                return jnp.logical_and(jnp.logical_not(hit), lo < hi)

            def obody(st):
                lo, hi, mid, _ = st
                tvec = _key_to_fvec(mid)
                nw = sweep_blocks(tvec)
                cap_ok = nw <= _BCAP
                cnt_v[...] = zero16

                @pl.when(cap_ok)
                def _():
                    c2v, mxv = exact_compact(tvec, nw)
                    cnt_v[...] = c2v
                    bk = plsc.bitcast(mxv, jnp.int32)
                    kk = jnp.where(bk >= 0, bk, _IMIN - bk)
                    key_v[...] = jnp.minimum(jnp.full((16,), hi, jnp.int32),
                                             kk)

                c2 = jnp.max(cnt_v[...])
                ge = jnp.logical_or(jnp.logical_not(cap_ok), c2 >= _TOPK)
                hit = jnp.logical_and(cap_ok, c2 >= _TOPK)
                lo2 = jnp.where(ge, mid, lo)
                hi2 = jnp.where(ge, hi, mid - 1)
                mid2 = jnp.where(hit, mid, _ceil_avg(lo2, hi2))
                return (lo2, hi2, mid2, hit)

            mid0 = jnp.clip(warm, jnp.int32(_KEY_LO + 1), jnp.int32(_KEY_HI))
            st0 = (jnp.int32(_KEY_LO), jnp.int32(_KEY_HI), mid0,
                   jnp.bool_(False))
            lo, hi, mid, hit = lax.while_loop(cond, obody, st0)

            @pl.when(jnp.logical_not(hit))
            def _():
                cnt_v[...] = strict_compact(_key_to_fvec(lo))
                key_v[...] = jnp.full((16,), lo, jnp.int32)

            c2 = jnp.max(cnt_v[...])
            hi2 = jnp.max(key_v[...])
            lo2 = jnp.where(hit, mid, lo)

            for w in range(4):
                plsc.store_scatter(
                    blk_v,
                    [jnp.full((16,), c2 + 16 * w, jnp.int32) + iota],
                    sent_vec)
            ng2 = (c2 + 63) // 64
            nv2 = (c2 + 15) // 16

            def rcond(st):
                rlo, rhi = st
                return rlo < rhi
            def rbody(st):
                rlo, rhi = st
                rmid = _ceil_avg(rlo, rhi)
                rc = count_ref(blk_v, ng2, _key_to_fvec(rmid))
                return (jnp.where(rc >= _TOPK, rmid, rlo),
                        jnp.where(rc >= _TOPK, rhi, rmid - 1))
            t_key, _ = lax.while_loop(rcond, rbody, (lo2, hi2))

            tvec = _key_to_fvec(t_key)
            def fbody(v, _):
                out_v[pl.ds(v * 16, 16)] = tvec
                return 0
            lax.fori_loop(0, _TOPK // 16, fbody, 0)

            def ebody(i, off):
                xv = blk_v[pl.ds(i * 16, 16)]
                m = xv > tvec
                cs = plsc.cumsum(jnp.where(m, jnp.int32(1), jnp.int32(0)))
                idx = jnp.minimum(off + cs - 1, jnp.int32(_TOPK - 1))
                plsc.store_scatter(out_v, [idx], xv, mask=m)
                return off + plsc.all_reduce_population_count(m)
            lax.fori_loop(0, nv2, ebody, zero16)

            pltpu.sync_copy(out_v, outneg_hbm.at[r])
            return t_key

        def prep_row(j):
            jv = jnp.full((16,), j, jnp.int32)
            tg = plsc.load_gather(tgt_v, [jv])
            xpos = plsc.load_gather(row_v, [tg])
            plsc.store_scatter(pos_v, [jv], xpos, mask=lane0)
            plsc.store_scatter(row_v, [tg], sent_vec, mask=lane0)

        pltpu.sync_copy(tgt_hbm.at[pl.ds(base, rpw)], tgt_v)

        pltpu.sync_copy(x_hbm.at[base], row_v)
        prep_row(jnp.int32(0))
        t0 = process_row(base, bootstrap())

        def rowbody(j, warm):
            pltpu.sync_copy(x_hbm.at[base + j], row_v)
            prep_row(j)
            t = process_row(base + j, warm)
            return t - _NUDGE

        lax.fori_loop(1, rpw, rowbody, t0 - _NUDGE)
        pltpu.sync_copy(pos_v, outpos_hbm.at[pl.ds(base, rpw)])

    return sc_kernel(x, tgt)


def _f_neg(v):
    s = 1.0 / (1.0 + jnp.exp(-v))
    return jnp.log(1.0 - jnp.minimum(s, jnp.float32(0.999)))


def _tc_reduce_body(neg_ref, pos_ref, out_ref):
    neg_sum = jnp.sum(_f_neg(neg_ref[...]), keepdims=True)
    p = jnp.maximum(1.0 / (1.0 + jnp.exp(-pos_ref[...])), jnp.float32(0.001))
    pos_sum = jnp.sum(jnp.log(p), keepdims=True)
    b = pos_ref.shape[0] * pos_ref.shape[1]
    out_ref[...] = (_GAMMA * (-pos_sum / b)
                    + (1.0 - _GAMMA) * (-neg_sum / (b * _TOPK)))


def kernel(input, target):
    b, c = input.shape
    neg_tops, pos_logits = _sc_topk(input, target.astype(jnp.int32))
    loss = pl.pallas_call(
        _tc_reduce_body,
        out_shape=jax.ShapeDtypeStruct((1, 1), jnp.float32),
    )(neg_tops, pos_logits.reshape(8, b // 8))
    return loss[0, 0]


# speedup vs baseline: 101.2259x; 1.2406x over previous
import functools

import jax
import jax.numpy as jnp
from jax import lax
from jax.experimental import pallas as pl
from jax.experimental.pallas import tpu as pltpu
from jax.experimental.pallas import tpu_sc as plsc

_GAMMA = 0.5
_TOPK = 512
_NCORE = 2
_NSUB = 16
_NW = _NCORE * _NSUB
_CAP = 1024
_BCAP = 16 * _CAP
_SENT = -3.0e38
_IMIN = -(2 ** 31)
_KEY_LO = -2139095040
_KEY_HI = 2139095039
_SUB_NV = 256
_SUB_RANK = 48
_NUDGE = 1 << 19
_U = 10


def _ceil_avg(lo, hi):
    return (lo >> 1) + (hi >> 1) + (lo & hi & 1) + ((lo ^ hi) & 1)


def _key_to_fvec(mid):
    mv = jnp.full((16,), mid, jnp.int32)
    bv = jnp.where(mv >= 0, mv, _IMIN - mv)
    return plsc.bitcast(bv, jnp.float32)


def _sc_topk(x, tgt):
    b, c = x.shape
    nv = c // 16
    rpw = b // _NW
    mesh = plsc.VectorSubcoreMesh(core_axis_name="c", subcore_axis_name="s")

    @functools.partial(
        pl.kernel,
        out_type=[
            jax.ShapeDtypeStruct((b, _TOPK), jnp.float32),
            jax.ShapeDtypeStruct((b,), jnp.float32),
        ],
        mesh=mesh,
        compiler_params=pltpu.CompilerParams(needs_layout_passes=False),
        scratch_types=[
            pltpu.VMEM((c,), jnp.float32),
            pltpu.VMEM((_BCAP + 80,), jnp.float32),
            pltpu.VMEM((_TOPK,), jnp.float32),
            pltpu.VMEM((rpw,), jnp.int32),
            pltpu.VMEM((rpw,), jnp.float32),
            pltpu.VMEM((16,), jnp.int32),
            pltpu.VMEM((16,), jnp.int32),
            pltpu.SemaphoreType.DMA,
        ],
    )
    def sc_kernel(x_hbm, tgt_hbm, outneg_hbm, outpos_hbm,
                  row_v, blk_v, out_v, tgt_v, pos_v, cnt_v, key_v, dma_sem):
        wid = lax.axis_index("s") * _NCORE + lax.axis_index("c")
        base = wid * rpw
        iota = lax.iota(jnp.int32, 16)
        lane0 = iota == 0
        zero16 = jnp.zeros((16,), jnp.int32)
        sent_vec = jnp.full((16,), _SENT, jnp.float32)

        def count_ref(ref, ngroups, tvec):
            def cbody(i, acc):
                ps = []
                for u in range(4):
                    xv = ref[pl.ds((i * 4 + u) * 16, 16)]
                    ps.append(plsc.all_reduce_population_count(xv >= tvec))
                return acc + ((ps[0] + ps[1]) + (ps[2] + ps[3]))
            return jnp.max(lax.fori_loop(0, ngroups, cbody, zero16))

        def sweep_blocks(tvec):
            def sbody(it, off):
                xs, advs = [], []
                for u in range(_U):
                    xv = row_v[pl.ds((it * _U + u) * 16, 16)]
                    m = xv >= tvec
                    pc = plsc.all_reduce_population_count(m)
                    xs.append(xv)
                    advs.append(jnp.where(pc > 0, jnp.int32(16), jnp.int32(0)))
                for u in range(_U):
                    idx = (off & jnp.int32(_BCAP - 1)) + iota
                    plsc.store_scatter(blk_v, [idx], xs[u])
                    off = off + advs[u]
                return off
            off = lax.fori_loop(0, nv // _U, sbody, zero16)
            return jnp.max(off)

        def exact_compact(tvec, nwords):
            def p2(k, carry):
                off2, mx = carry
                xs, ms, css, pcs = [], [], [], []
                for u in range(4):
                    xv = blk_v[pl.ds((k * 4 + u) * 16, 16)]
                    m = xv >= tvec
                    css.append(plsc.cumsum(
                        jnp.where(m, jnp.int32(1), jnp.int32(0))))
                    pcs.append(plsc.all_reduce_population_count(m))
                    mx = jnp.maximum(mx, jnp.where(m, xv, sent_vec))
                    xs.append(xv)
                    ms.append(m)
                for u in range(4):
                    plsc.store_scatter(blk_v, [off2 + css[u] - 1], xs[u],
                                       mask=ms[u])
                    off2 = off2 + pcs[u]
                return off2, mx
            return lax.fori_loop(0, (nwords // 16 + 3) // 4, p2,
                                 (zero16, sent_vec))

        def strict_compact(tvec):
            def sbody(i, off):
                xv = row_v[pl.ds(i * 16, 16)]
                m = xv > tvec
                cs = plsc.cumsum(jnp.where(m, jnp.int32(1), jnp.int32(0)))
                idx = jnp.minimum(off + cs - 1, jnp.int32(_BCAP + 15))
                plsc.store_scatter(blk_v, [idx], xv, mask=m)
                return off + plsc.all_reduce_population_count(m)
            return lax.fori_loop(0, nv, sbody, zero16)

        def bootstrap():
            def cond(st):
                lo, hi = st
                return lo < hi
            def bbody(st):
                lo, hi = st
                mid = _ceil_avg(lo, hi)
                cc = count_ref(row_v, _SUB_NV // 4, _key_to_fvec(mid))
                return (jnp.where(cc >= _SUB_RANK, mid, lo),
                        jnp.where(cc >= _SUB_RANK, hi, mid - 1))
            lo, _ = lax.while_loop(
                cond, bbody, (jnp.int32(_KEY_LO), jnp.int32(_KEY_HI)))
            return lo

        def process_row(r, warm, fetch_next):
            def cond(st):
                lo, hi, mid, hit = st
                return jnp.logical_and(jnp.logical_not(hit), lo < hi)

            def obody(st):
                lo, hi, mid, _ = st
                tvec = _key_to_fvec(mid)
                nw = sweep_blocks(tvec)
                cap_ok = nw <= _BCAP
                cnt_v[...] = zero16

                @pl.when(cap_ok)
                def _():
                    for w in range(4):
                        plsc.store_scatter(
                            blk_v,
                            [jnp.full((16,), nw + 16 * w, jnp.int32) + iota],
                            sent_vec)
                    c2v, mxv = exact_compact(tvec, nw)
                    cnt_v[...] = c2v
                    bk = plsc.bitcast(mxv, jnp.int32)
                    kk = jnp.where(bk >= 0, bk, _IMIN - bk)
                    key_v[...] = jnp.minimum(jnp.full((16,), hi, jnp.int32),
                                             kk)

                c2 = jnp.max(cnt_v[...])
                ge = jnp.logical_or(jnp.logical_not(cap_ok), c2 >= _TOPK)
                hit = jnp.logical_and(cap_ok, c2 >= _TOPK)
                lo2 = jnp.where(ge, mid, lo)
                hi2 = jnp.where(ge, hi, mid - 1)
                mid2 = jnp.where(hit, mid, _ceil_avg(lo2, hi2))
                return (lo2, hi2, mid2, hit)

            mid0 = jnp.clip(warm, jnp.int32(_KEY_LO + 1), jnp.int32(_KEY_HI))
            st0 = (jnp.int32(_KEY_LO), jnp.int32(_KEY_HI), mid0,
                   jnp.bool_(False))
            lo, hi, mid, hit = lax.while_loop(cond, obody, st0)

            @pl.when(jnp.logical_not(hit))
            def _():
                cnt_v[...] = strict_compact(_key_to_fvec(lo))
                key_v[...] = jnp.full((16,), lo, jnp.int32)

            @pl.when(fetch_next)
            def _():
                pltpu.async_copy(x_hbm.at[r + 1], row_v, dma_sem)

            c2 = jnp.max(cnt_v[...])
            hi2 = jnp.max(key_v[...])
            lo2 = jnp.where(hit, mid, lo)

            for w in range(4):
                plsc.store_scatter(
                    blk_v,
                    [jnp.full((16,), c2 + 16 * w, jnp.int32) + iota],
                    sent_vec)
            ng2 = (c2 + 63) // 64
            nv2 = (c2 + 15) // 16

            def rcond(st):
                rlo, rhi = st
                return rlo < rhi
            def rbody(st):
                rlo, rhi = st
                rmid = _ceil_avg(rlo, rhi)
                rc = count_ref(blk_v, ng2, _key_to_fvec(rmid))
                return (jnp.where(rc >= _TOPK, rmid, rlo),
                        jnp.where(rc >= _TOPK, rhi, rmid - 1))
            t_key, _ = lax.while_loop(rcond, rbody, (lo2, hi2))

            tvec = _key_to_fvec(t_key)
            def fbody(v, _):
                out_v[pl.ds(v * 16, 16)] = tvec
                return 0
            lax.fori_loop(0, _TOPK // 16, fbody, 0)

            def ebody(i, off):
                xv = blk_v[pl.ds(i * 16, 16)]
                m = xv > tvec
                cs = plsc.cumsum(jnp.where(m, jnp.int32(1), jnp.int32(0)))
                idx = jnp.minimum(off + cs - 1, jnp.int32(_TOPK - 1))
                plsc.store_scatter(out_v, [idx], xv, mask=m)
                return off + plsc.all_reduce_population_count(m)
            lax.fori_loop(0, nv2, ebody, zero16)

            pltpu.sync_copy(out_v, outneg_hbm.at[r])
            return t_key

        def prep_row(j):
            jv = jnp.full((16,), j, jnp.int32)
            tg = plsc.load_gather(tgt_v, [jv])
            xpos = plsc.load_gather(row_v, [tg])
            plsc.store_scatter(pos_v, [jv], xpos, mask=lane0)
            plsc.store_scatter(row_v, [tg], sent_vec, mask=lane0)

        pltpu.sync_copy(tgt_hbm.at[pl.ds(base, rpw)], tgt_v)

        pltpu.sync_copy(x_hbm.at[base], row_v)
        prep_row(jnp.int32(0))
        t0 = process_row(base, bootstrap(), jnp.bool_(rpw > 1))

        def rowbody(j, warm):
            r = base + j
            pltpu.make_async_copy(x_hbm.at[r], row_v, dma_sem).wait()
            prep_row(j)
            t = process_row(r, warm, j < rpw - 1)
            return t - _NUDGE

        lax.fori_loop(1, rpw, rowbody, t0 - _NUDGE)
        pltpu.sync_copy(pos_v, outpos_hbm.at[pl.ds(base, rpw)])

    return sc_kernel(x, tgt)


def _f_neg(v):
    s = 1.0 / (1.0 + jnp.exp(-v))
    return jnp.log(1.0 - jnp.minimum(s, jnp.float32(0.999)))


def _tc_reduce_body(neg_ref, pos_ref, out_ref):
    neg_sum = jnp.sum(_f_neg(neg_ref[...]), keepdims=True)
    p = jnp.maximum(1.0 / (1.0 + jnp.exp(-pos_ref[...])), jnp.float32(0.001))
    pos_sum = jnp.sum(jnp.log(p), keepdims=True)
    b = pos_ref.shape[0] * pos_ref.shape[1]
    out_ref[...] = (_GAMMA * (-pos_sum / b)
                    + (1.0 - _GAMMA) * (-neg_sum / (b * _TOPK)))


def kernel(input, target):
    b, c = input.shape
    neg_tops, pos_logits = _sc_topk(input, target.astype(jnp.int32))
    loss = pl.pallas_call(
        _tc_reduce_body,
        out_shape=jax.ShapeDtypeStruct((1, 1), jnp.float32),
    )(neg_tops, pos_logits.reshape(8, b // 8))
    return loss[0, 0]


# speedup vs baseline: 111.2282x; 1.0988x over previous
import functools

import jax
import jax.numpy as jnp
from jax import lax
from jax.experimental import pallas as pl
from jax.experimental.pallas import tpu as pltpu
from jax.experimental.pallas import tpu_sc as plsc

_GAMMA = 0.5
_TOPK = 512
_NCORE = 2
_NSUB = 16
_NW = _NCORE * _NSUB
_CAP = 1024
_BCAP = 16 * _CAP
_SENT = -3.0e38
_IMIN = -(2 ** 31)
_KEY_LO = -2139095040
_KEY_HI = 2139095039
_SUB_NV = 256
_SUB_RANK = 48
_NUDGE = 1 << 19
_U = 25


def _ceil_avg(lo, hi):
    return (lo >> 1) + (hi >> 1) + (lo & hi & 1) + ((lo ^ hi) & 1)


def _key_to_fvec(mid):
    mv = jnp.full((16,), mid, jnp.int32)
    bv = jnp.where(mv >= 0, mv, _IMIN - mv)
    return plsc.bitcast(bv, jnp.float32)


def _sc_topk(x, tgt):
    b, c = x.shape
    nv = c // 16
    rpw = b // _NW
    mesh = plsc.VectorSubcoreMesh(core_axis_name="c", subcore_axis_name="s")

    @functools.partial(
        pl.kernel,
        out_type=[
            jax.ShapeDtypeStruct((b, _TOPK), jnp.float32),
            jax.ShapeDtypeStruct((b,), jnp.float32),
        ],
        mesh=mesh,
        compiler_params=pltpu.CompilerParams(needs_layout_passes=False),
        scratch_types=[
            pltpu.VMEM((c,), jnp.float32),
            pltpu.VMEM((_BCAP + 80,), jnp.float32),
            pltpu.VMEM((_TOPK,), jnp.float32),
            pltpu.VMEM((rpw,), jnp.int32),
            pltpu.VMEM((rpw,), jnp.float32),
            pltpu.VMEM((16,), jnp.int32),
            pltpu.VMEM((16,), jnp.int32),
            pltpu.SemaphoreType.DMA,
            pltpu.SemaphoreType.DMA,
        ],
    )
    def sc_kernel(x_hbm, tgt_hbm, outneg_hbm, outpos_hbm,
                  row_v, blk_v, out_v, tgt_v, pos_v, cnt_v, key_v,
                  dma_sem, out_sem):
        wid = lax.axis_index("s") * _NCORE + lax.axis_index("c")
        base = wid * rpw
        iota = lax.iota(jnp.int32, 16)
        lane0 = iota == 0
        zero16 = jnp.zeros((16,), jnp.int32)
        sent_vec = jnp.full((16,), _SENT, jnp.float32)

        def count_ref(ref, ngroups, tvec):
            def cbody(i, acc):
                ps = []
                for u in range(4):
                    xv = ref[pl.ds((i * 4 + u) * 16, 16)]
                    ps.append(plsc.all_reduce_population_count(xv >= tvec))
                return acc + ((ps[0] + ps[1]) + (ps[2] + ps[3]))
            return jnp.max(lax.fori_loop(0, ngroups, cbody, zero16))

        def sweep_blocks(tvec):
            def sbody(it, off):
                xs, advs = [], []
                for u in range(_U):
                    xv = row_v[pl.ds((it * _U + u) * 16, 16)]
                    m = xv >= tvec
                    pc = plsc.all_reduce_population_count(m)
                    xs.append(xv)
                    advs.append(jnp.where(pc > 0, jnp.int32(16), jnp.int32(0)))
                for u in range(_U):
                    idx = (off & jnp.int32(_BCAP - 1)) + iota
                    plsc.store_scatter(blk_v, [idx], xs[u])
                    off = off + advs[u]
                return off
            off = lax.fori_loop(0, nv // _U, sbody, zero16)
            return jnp.max(off)

        def exact_compact(tvec, nwords):
            def p2(k, carry):
                off2, mx = carry
                xs, ms, css, pcs = [], [], [], []
                for u in range(4):
                    xv = blk_v[pl.ds((k * 4 + u) * 16, 16)]
                    m = xv >= tvec
                    css.append(plsc.cumsum(
                        jnp.where(m, jnp.int32(1), jnp.int32(0))))
                    pcs.append(plsc.all_reduce_population_count(m))
                    mx = jnp.maximum(mx, jnp.where(m, xv, sent_vec))
                    xs.append(xv)
                    ms.append(m)
                for u in range(4):
                    plsc.store_scatter(blk_v, [off2 + css[u] - 1], xs[u],
                                       mask=ms[u])
                    off2 = off2 + pcs[u]
                return off2, mx
            return lax.fori_loop(0, (nwords // 16 + 3) // 4, p2,
                                 (zero16, sent_vec))

        def strict_compact(tvec):
            def sbody(i, off):
                xv = row_v[pl.ds(i * 16, 16)]
                m = xv > tvec
                cs = plsc.cumsum(jnp.where(m, jnp.int32(1), jnp.int32(0)))
                idx = jnp.minimum(off + cs - 1, jnp.int32(_BCAP + 15))
                plsc.store_scatter(blk_v, [idx], xv, mask=m)
                return off + plsc.all_reduce_population_count(m)
            return lax.fori_loop(0, nv, sbody, zero16)

        def bootstrap():
            def cond(st):
                lo, hi = st
                return lo < hi
            def bbody(st):
                lo, hi = st
                mid = _ceil_avg(lo, hi)
                cc = count_ref(row_v, _SUB_NV // 4, _key_to_fvec(mid))
                return (jnp.where(cc >= _SUB_RANK, mid, lo),
                        jnp.where(cc >= _SUB_RANK, hi, mid - 1))
            lo, _ = lax.while_loop(
                cond, bbody, (jnp.int32(_KEY_LO), jnp.int32(_KEY_HI)))
            return lo

        def process_row(r, warm, fetch_next):
            def cond(st):
                lo, hi, mid, hit = st
                return jnp.logical_and(jnp.logical_not(hit), lo < hi)

            def obody(st):
                lo, hi, mid, _ = st
                tvec = _key_to_fvec(mid)
                nw = sweep_blocks(tvec)
                cap_ok = nw <= _BCAP
                cnt_v[...] = zero16

                @pl.when(cap_ok)
                def _():
                    for w in range(4):
                        plsc.store_scatter(
                            blk_v,
                            [jnp.full((16,), nw + 16 * w, jnp.int32) + iota],
                            sent_vec)
                    c2v, mxv = exact_compact(tvec, nw)
                    cnt_v[...] = c2v
                    bk = plsc.bitcast(mxv, jnp.int32)
                    kk = jnp.where(bk >= 0, bk, _IMIN - bk)
                    key_v[...] = jnp.minimum(jnp.full((16,), hi, jnp.int32),
                                             kk)

                c2 = jnp.max(cnt_v[...])
                ge = jnp.logical_or(jnp.logical_not(cap_ok), c2 >= _TOPK)
                hit = jnp.logical_and(cap_ok, c2 >= _TOPK)
                lo2 = jnp.where(ge, mid, lo)
                hi2 = jnp.where(ge, hi, mid - 1)
                mid2 = jnp.where(hit, mid, _ceil_avg(lo2, hi2))
                return (lo2, hi2, mid2, hit)

            mid0 = jnp.clip(warm, jnp.int32(_KEY_LO + 1), jnp.int32(_KEY_HI))
            st0 = (jnp.int32(_KEY_LO), jnp.int32(_KEY_HI), mid0,
                   jnp.bool_(False))
            lo, hi, mid, hit = lax.while_loop(cond, obody, st0)

            @pl.when(jnp.logical_not(hit))
            def _():
                cnt_v[...] = strict_compact(_key_to_fvec(lo))
                key_v[...] = jnp.full((16,), lo, jnp.int32)

            @pl.when(fetch_next)
            def _():
                pltpu.async_copy(x_hbm.at[r + 1], row_v, dma_sem)

            c2 = jnp.max(cnt_v[...])
            hi2 = jnp.max(key_v[...])
            lo2 = jnp.where(hit, mid, lo)

            for w in range(4):
                plsc.store_scatter(
                    blk_v,
                    [jnp.full((16,), c2 + 16 * w, jnp.int32) + iota],
                    sent_vec)
            ng2 = (c2 + 63) // 64
            nv2 = (c2 + 15) // 16

            def rcond(st):
                rlo, rhi = st
                return rlo < rhi
            def rbody(st):
                rlo, rhi = st
                rmid = _ceil_avg(rlo, rhi)
                rc = count_ref(blk_v, ng2, _key_to_fvec(rmid))
                return (jnp.where(rc >= _TOPK, rmid, rlo),
                        jnp.where(rc >= _TOPK, rhi, rmid - 1))
            t_key, _ = lax.while_loop(rcond, rbody, (lo2, hi2))

            @pl.when(r > base)
            def _():
                pltpu.make_async_copy(out_v, outneg_hbm.at[r], out_sem).wait()

            tvec = _key_to_fvec(t_key)
            def fbody(v, _):
                out_v[pl.ds(v * 16, 16)] = tvec
                return 0
            lax.fori_loop(0, _TOPK // 16, fbody, 0)

            def ebody(i, off):
                xv = blk_v[pl.ds(i * 16, 16)]
                m = xv > tvec
                cs = plsc.cumsum(jnp.where(m, jnp.int32(1), jnp.int32(0)))
                idx = jnp.minimum(off + cs - 1, jnp.int32(_TOPK - 1))
                plsc.store_scatter(out_v, [idx], xv, mask=m)
                return off + plsc.all_reduce_population_count(m)
            lax.fori_loop(0, nv2, ebody, zero16)

            pltpu.async_copy(out_v, outneg_hbm.at[r], out_sem)
            return t_key

        def prep_row(j):
            jv = jnp.full((16,), j, jnp.int32)
            tg = plsc.load_gather(tgt_v, [jv])
            xpos = plsc.load_gather(row_v, [tg])
            plsc.store_scatter(pos_v, [jv], xpos, mask=lane0)
            plsc.store_scatter(row_v, [tg], sent_vec, mask=lane0)

        pltpu.sync_copy(tgt_hbm.at[pl.ds(base, rpw)], tgt_v)

        pltpu.sync_copy(x_hbm.at[base], row_v)
        prep_row(jnp.int32(0))
        t0 = process_row(base, bootstrap(), jnp.bool_(rpw > 1))

        def rowbody(j, warm):
            r = base + j
            pltpu.make_async_copy(x_hbm.at[r], row_v, dma_sem).wait()
            prep_row(j)
            t = process_row(r, warm, j < rpw - 1)
            return t - _NUDGE

        lax.fori_loop(1, rpw, rowbody, t0 - _NUDGE)
        pltpu.make_async_copy(out_v, outneg_hbm.at[base], out_sem).wait()
        pltpu.sync_copy(pos_v, outpos_hbm.at[pl.ds(base, rpw)])

    return sc_kernel(x, tgt)


def _f_neg(v):
    s = 1.0 / (1.0 + jnp.exp(-v))
    return jnp.log(1.0 - jnp.minimum(s, jnp.float32(0.999)))


def _tc_reduce_body(neg_ref, pos_ref, out_ref):
    neg_sum = jnp.sum(_f_neg(neg_ref[...]), keepdims=True)
    p = jnp.maximum(1.0 / (1.0 + jnp.exp(-pos_ref[...])), jnp.float32(0.001))
    pos_sum = jnp.sum(jnp.log(p), keepdims=True)
    b = pos_ref.shape[0] * pos_ref.shape[1]
    out_ref[...] = (_GAMMA * (-pos_sum / b)
                    + (1.0 - _GAMMA) * (-neg_sum / (b * _TOPK)))


def kernel(input, target):
    b, c = input.shape
    neg_tops, pos_logits = _sc_topk(input, target.astype(jnp.int32))
    loss = pl.pallas_call(
        _tc_reduce_body,
        out_shape=jax.ShapeDtypeStruct((1, 1), jnp.float32),
    )(neg_tops, pos_logits.reshape(8, b // 8))
    return loss[0, 0]
